# baseline (speedup 1.0000x reference)
import functools

import jax
import jax.numpy as jnp
from jax import lax
from jax.experimental import pallas as pl
from jax.experimental.pallas import tpu as pltpu
from jax.experimental.pallas import tpu_sc as plsc


def kernel(x, pos_emb):
    batch = x.shape[0]
    seq, dim = pos_emb.shape
    info = plsc.get_sparse_core_info()
    nworkers = info.num_cores * info.num_subcores
    b_per_w = batch // nworkers
    mesh = plsc.VectorSubcoreMesh(core_axis_name="c", subcore_axis_name="s")

    @functools.partial(
        pl.kernel,
        mesh=mesh,
        out_type=jax.ShapeDtypeStruct((batch, seq, dim), jnp.float32),
        scratch_types=[
            pltpu.VMEM((seq, dim), jnp.float32),
            pltpu.SemaphoreType.DMA,
        ],
    )
    def k(pos_hbm, out_hbm, tab_v, sem):
        wid = lax.axis_index("s") * info.num_cores + lax.axis_index("c")
        base = wid * b_per_w
        pltpu.sync_copy(pos_hbm, tab_v)
        for b in range(b_per_w):
            pltpu.make_async_copy(tab_v, out_hbm.at[base + b], sem).start()
        for b in range(b_per_w):
            pltpu.make_async_copy(tab_v, out_hbm.at[base + b], sem).wait()

    return k(pos_emb)


# speedup vs baseline: 1.4685x; 1.4685x over previous
import jax
import jax.numpy as jnp
from jax.experimental import pallas as pl


_BATCH_BLOCK = 16


def _broadcast_body(pos_emb_ref, out_ref):
    out_ref[...] = jnp.broadcast_to(pos_emb_ref[...][None], out_ref.shape)


def kernel(x, pos_emb):
    batch = x.shape[0]
    seq, dim = pos_emb.shape
    grid = batch // _BATCH_BLOCK
    return pl.pallas_call(
        _broadcast_body,
        grid=(grid,),
        in_specs=[pl.BlockSpec((seq, dim), lambda i: (0, 0))],
        out_specs=pl.BlockSpec((_BATCH_BLOCK, seq, dim), lambda i: (i, 0, 0)),
        out_shape=jax.ShapeDtypeStruct((batch, seq, dim), jnp.float32),
    )(pos_emb)


# speedup vs baseline: 1.8962x; 1.2912x over previous
import jax
import jax.numpy as jnp
from jax.experimental import pallas as pl


_BATCH_BLOCK = 32


def _broadcast_body(pos_emb_ref, out_ref):
    out_ref[...] = jnp.broadcast_to(pos_emb_ref[...][None], out_ref.shape)


def kernel(x, pos_emb):
    batch = x.shape[0]
    seq, dim = pos_emb.shape
    grid = batch // _BATCH_BLOCK
    return pl.pallas_call(
        _broadcast_body,
        grid=(grid,),
        in_specs=[pl.BlockSpec((seq, dim), lambda i: (0, 0))],
        out_specs=pl.BlockSpec((_BATCH_BLOCK, seq, dim), lambda i: (i, 0, 0)),
        out_shape=jax.ShapeDtypeStruct((batch, seq, dim), jnp.float32),
    )(pos_emb)
